# Initial kernel scaffold; baseline (speedup 1.0000x reference)
import jax
import jax.numpy as jnp
from jax.experimental import pallas as pl


def kernel(x, Wr, W1, b1, W2, b2):
    raise NotImplementedError("write your pallas kernel here")


# speedup vs baseline: 3.0882x; 3.0882x over previous
import functools

import jax
import jax.numpy as jnp
from jax import lax
from jax.experimental import pallas as pl
from jax.experimental.pallas import tpu as pltpu
from jax.experimental.pallas import tpu_sc as plsc

DIM = 2048
E = 8
DFF = 8192
N = 8192
CAP = 1280
R = E * CAP
NW = 32
TPW = N // NW
CHUNK = 16
NCH = TPW // CHUNK
CCH = 8
NCC = TPW // CCH
RPAD = R + NW
FBLK = 256
LANES = 16

_sc_mesh = plsc.VectorSubcoreMesh(core_axis_name="c", subcore_axis_name="s")


def _router_body(x_ref, wr_ref, out_ref):
    lg = lax.dot_general(x_ref[...], wr_ref[...], (((1,), (1,)), ((), ())),
                         preferred_element_type=jnp.float32)
    lane = lax.broadcasted_iota(jnp.int32, lg.shape, 1)
    out_ref[...] = jnp.where(lane < E, lg, -1e30)


def _router(xf, wr_pad):
    tb = 1024
    return pl.pallas_call(
        _router_body,
        grid=(N // tb,),
        in_specs=[pl.BlockSpec((tb, DIM), lambda i: (i, 0)),
                  pl.BlockSpec((128, DIM), lambda i: (0, 0))],
        out_specs=pl.BlockSpec((tb, 128), lambda i: (i, 0)),
        out_shape=jax.ShapeDtypeStruct((N, 128), jnp.float32),
    )(xf, wr_pad)


@functools.partial(
    pl.kernel,
    out_type=jax.ShapeDtypeStruct((RPAD, DIM), jnp.float32),
    mesh=_sc_mesh,
    scratch_types=[pltpu.VMEM((NCH, CHUNK), jnp.int32),
                   pltpu.VMEM((CHUNK, DIM), jnp.float32),
                   pltpu.SemaphoreType.DMA],
)
def _dispatch(xf_hbm, slot_hbm, buf_hbm, idx_v, rows_v, sem):
    wid = lax.axis_index("s") * 2 + lax.axis_index("c")
    pltpu.sync_copy(slot_hbm.at[pl.ds(wid * NCH, NCH)], idx_v)
    for c in range(NCH):
        base = wid * TPW + c * CHUNK
        pltpu.sync_copy(xf_hbm.at[pl.ds(base, CHUNK)], rows_v)
        pltpu.async_copy(rows_v, buf_hbm.at[idx_v.at[c]], sem).wait()


def _ffn_body(x_ref, w1_ref, b1_ref, w2_ref, b2_ref, out_ref):
    f = pl.program_id(1)

    @pl.when(f == 0)
    def _():
        out_ref[...] = jnp.broadcast_to(b2_ref[0], out_ref.shape)

    h = jnp.dot(x_ref[...], w1_ref[0], preferred_element_type=jnp.float32)
    h = jax.nn.gelu(h + b1_ref[0])
    out_ref[...] += jnp.dot(h, w2_ref[0], preferred_element_type=jnp.float32)


def _ffn(buf, w1, b1, w2, b2):
    grid = (E, DFF // FBLK)
    return pl.pallas_call(
        _ffn_body,
        grid=grid,
        in_specs=[
            pl.BlockSpec((CAP, DIM), lambda e, f: (e, 0)),
            pl.BlockSpec((1, DIM, FBLK), lambda e, f: (e, 0, f)),
            pl.BlockSpec((1, 1, FBLK), lambda e, f: (e, 0, f)),
            pl.BlockSpec((1, FBLK, DIM), lambda e, f: (e, f, 0)),
            pl.BlockSpec((1, 1, DIM), lambda e, f: (e, 0, 0)),
        ],
        out_specs=pl.BlockSpec((CAP, DIM), lambda e, f: (e, 0)),
        out_shape=jax.ShapeDtypeStruct((R, DIM), jnp.float32),
        compiler_params=pltpu.CompilerParams(
            dimension_semantics=("parallel", "arbitrary")),
    )(buf, w1, b1, w2, b2)


@functools.partial(
    pl.kernel,
    out_type=jax.ShapeDtypeStruct((N, DIM), jnp.float32),
    mesh=_sc_mesh,
    scratch_types=[pltpu.VMEM((NCC, CCH), jnp.int32),
                   pltpu.VMEM((TPW, LANES), jnp.float32),
                   pltpu.VMEM((TPW, LANES), jnp.int32),
                   pltpu.VMEM((CCH, DIM), jnp.float32),
                   pltpu.VMEM((CCH, DIM), jnp.float32),
                   pltpu.SemaphoreType.DMA],
)
def _combine(eo_hbm, xf_hbm, gslot_hbm, grep_hbm, krep_hbm, out_hbm,
             idx_v, g_v, k_v, eo_v, xf_v, sem):
    wid = lax.axis_index("s") * 2 + lax.axis_index("c")
    pltpu.sync_copy(gslot_hbm.at[pl.ds(wid * NCC, NCC)], idx_v)
    pltpu.sync_copy(grep_hbm.at[pl.ds(wid * TPW, TPW)], g_v)
    pltpu.sync_copy(krep_hbm.at[pl.ds(wid * TPW, TPW)], k_v)
    for c in range(NCC):
        base = wid * TPW + c * CCH
        pltpu.async_copy(eo_hbm.at[idx_v.at[c]], eo_v, sem).wait()
        pltpu.sync_copy(xf_hbm.at[pl.ds(base, CCH)], xf_v)

        def tok_body(j, carry):
            g16 = g_v[c * CCH + j]
            k16 = k_v[c * CCH + j] > 0

            def col_body(col, inner):
                sl = pl.ds(col * LANES, LANES)
                e16 = eo_v[j, sl]
                x16 = xf_v[j, sl]
                xf_v[j, sl] = jnp.where(k16, g16 * e16, x16)
                return inner

            lax.fori_loop(0, DIM // LANES, col_body, 0, unroll=8)
            return carry

        lax.fori_loop(0, CCH, tok_body, 0)
        pltpu.sync_copy(xf_v, out_hbm.at[pl.ds(base, CCH)])


def kernel(x, Wr, W1, b1, W2, b2):
    xf = x.reshape(N, DIM)

    wr_pad = jnp.zeros((128, DIM), jnp.float32).at[:E].set(Wr)
    logits = _router(xf, wr_pad)
    probs = jax.nn.softmax(logits, axis=-1)[:, :E]
    gating = jnp.max(probs, axis=-1)
    idx = jnp.argmax(probs, axis=-1).astype(jnp.int32)

    mask = jax.nn.one_hot(idx, E, dtype=jnp.int32)
    pos_all = jnp.cumsum(mask, axis=0) - 1
    pos = jnp.sum(pos_all * mask, axis=1)
    kept = pos < CAP
    tok = jnp.arange(N, dtype=jnp.int32)
    slot = idx * CAP + pos
    slot_scatter = jnp.where(kept, slot, R + tok // TPW)
    slot_gather = jnp.where(kept, slot, 0)

    counts = jnp.sum(mask, axis=0)
    tokens_per_expert = jnp.minimum(counts, CAP).astype(jnp.float32)
    frac_tok = tokens_per_expert / N
    frac_prob = probs.mean(axis=0)
    aux = 0.01 * E * jnp.sum(frac_tok * frac_prob)

    buf = _dispatch(xf, slot_scatter.reshape(N // CHUNK, CHUNK))
    eo = _ffn(buf, W1, b1.reshape(E, 1, DFF), W2, b2.reshape(E, 1, DIM))

    gate_kept = jnp.where(kept, gating, 0.0)
    grep = jnp.broadcast_to(gate_kept[:, None], (N, LANES))
    krep = jnp.broadcast_to(kept.astype(jnp.int32)[:, None], (N, LANES))
    out = _combine(eo, xf, slot_gather.reshape(N // CCH, CCH), grep, krep)
    return out.reshape(x.shape), aux


# speedup vs baseline: 4.7952x; 1.5528x over previous
import functools

import jax
import jax.numpy as jnp
from jax import lax
from jax.experimental import pallas as pl
from jax.experimental.pallas import tpu as pltpu
from jax.experimental.pallas import tpu_sc as plsc

DIM = 2048
E = 8
DFF = 8192
N = 8192
CAP = 1280
R = E * CAP
NW = 32
TPW = N // NW
CHUNK = 16
NCH = TPW // CHUNK
CCH = 8
NCC = TPW // CCH
RPAD = R + NW
FBLK = 512
LANES = 16

_sc_mesh = plsc.VectorSubcoreMesh(core_axis_name="c", subcore_axis_name="s")


def _router_body(x_ref, wr_ref, out_ref):
    lg = lax.dot_general(x_ref[...], wr_ref[...], (((1,), (1,)), ((), ())),
                         preferred_element_type=jnp.float32)
    lane = lax.broadcasted_iota(jnp.int32, lg.shape, 1)
    out_ref[...] = jnp.where(lane < E, lg, -1e30)


def _router(xf, wr_pad):
    tb = 1024
    return pl.pallas_call(
        _router_body,
        grid=(N // tb,),
        in_specs=[pl.BlockSpec((tb, DIM), lambda i: (i, 0)),
                  pl.BlockSpec((128, DIM), lambda i: (0, 0))],
        out_specs=pl.BlockSpec((tb, 128), lambda i: (i, 0)),
        out_shape=jax.ShapeDtypeStruct((N, 128), jnp.float32),
    )(xf, wr_pad)


@functools.partial(
    pl.kernel,
    out_type=jax.ShapeDtypeStruct((RPAD, DIM), jnp.float32),
    mesh=_sc_mesh,
    scratch_types=[pltpu.VMEM((NCH, CHUNK), jnp.int32),
                   pltpu.VMEM((CHUNK, DIM), jnp.float32),
                   pltpu.SemaphoreType.DMA],
)
def _dispatch(xf_hbm, slot_hbm, buf_hbm, idx_v, rows_v, sem):
    wid = lax.axis_index("s") * 2 + lax.axis_index("c")
    pltpu.sync_copy(slot_hbm.at[pl.ds(wid * NCH, NCH)], idx_v)
    for c in range(NCH):
        base = wid * TPW + c * CHUNK
        pltpu.sync_copy(xf_hbm.at[pl.ds(base, CHUNK)], rows_v)
        pltpu.async_copy(rows_v, buf_hbm.at[idx_v.at[c]], sem).wait()


def _ffn_body(x_ref, w1_ref, b1_ref, w2_ref, b2_ref, out_ref):
    f = pl.program_id(1)

    @pl.when(f == 0)
    def _():
        out_ref[...] = jnp.broadcast_to(b2_ref[0], out_ref.shape)

    h = jnp.dot(x_ref[...], w1_ref[0], preferred_element_type=jnp.float32)
    h = jax.nn.gelu(h + b1_ref[0])
    out_ref[...] += jnp.dot(h, w2_ref[0], preferred_element_type=jnp.float32)


def _ffn(buf, w1, b1, w2, b2):
    grid = (E, DFF // FBLK)
    return pl.pallas_call(
        _ffn_body,
        grid=grid,
        in_specs=[
            pl.BlockSpec((CAP, DIM), lambda e, f: (e, 0)),
            pl.BlockSpec((1, DIM, FBLK), lambda e, f: (e, 0, f)),
            pl.BlockSpec((1, 1, FBLK), lambda e, f: (e, 0, f)),
            pl.BlockSpec((1, FBLK, DIM), lambda e, f: (e, f, 0)),
            pl.BlockSpec((1, 1, DIM), lambda e, f: (e, 0, 0)),
        ],
        out_specs=pl.BlockSpec((CAP, DIM), lambda e, f: (e, 0)),
        out_shape=jax.ShapeDtypeStruct((R, DIM), jnp.float32),
        compiler_params=pltpu.CompilerParams(
            dimension_semantics=("parallel", "arbitrary"),
            vmem_limit_bytes=63 * 1024 * 1024),
    )(buf, w1, b1, w2, b2)


@functools.partial(
    pl.kernel,
    out_type=jax.ShapeDtypeStruct((N, DIM), jnp.float32),
    mesh=_sc_mesh,
    scratch_types=[pltpu.VMEM((NCC, CCH), jnp.int32),
                   pltpu.VMEM((TPW, LANES), jnp.float32),
                   pltpu.VMEM((TPW, LANES), jnp.int32),
                   pltpu.VMEM((CCH, DIM), jnp.float32),
                   pltpu.VMEM((CCH, DIM), jnp.float32),
                   pltpu.SemaphoreType.DMA],
)
def _combine(eo_hbm, xf_hbm, gslot_hbm, grep_hbm, krep_hbm, out_hbm,
             idx_v, g_v, k_v, eo_v, xf_v, sem):
    wid = lax.axis_index("s") * 2 + lax.axis_index("c")
    pltpu.sync_copy(gslot_hbm.at[pl.ds(wid * NCC, NCC)], idx_v)
    pltpu.sync_copy(grep_hbm.at[pl.ds(wid * TPW, TPW)], g_v)
    pltpu.sync_copy(krep_hbm.at[pl.ds(wid * TPW, TPW)], k_v)
    for c in range(NCC):
        base = wid * TPW + c * CCH
        pltpu.async_copy(eo_hbm.at[idx_v.at[c]], eo_v, sem).wait()
        pltpu.sync_copy(xf_hbm.at[pl.ds(base, CCH)], xf_v)

        def tok_body(j, carry):
            g16 = g_v[c * CCH + j]
            k16 = k_v[c * CCH + j] > 0

            def col_body(col, inner):
                sl = pl.ds(col * LANES, LANES)
                e16 = eo_v[j, sl]
                x16 = xf_v[j, sl]
                xf_v[j, sl] = jnp.where(k16, g16 * e16, x16)
                return inner

            lax.fori_loop(0, DIM // LANES, col_body, 0, unroll=8)
            return carry

        lax.fori_loop(0, CCH, tok_body, 0)
        pltpu.sync_copy(xf_v, out_hbm.at[pl.ds(base, CCH)])


def kernel(x, Wr, W1, b1, W2, b2):
    xf = x.reshape(N, DIM)

    wr_pad = jnp.zeros((128, DIM), jnp.float32).at[:E].set(Wr)
    logits = _router(xf, wr_pad)
    probs = jax.nn.softmax(logits, axis=-1)[:, :E]
    gating = jnp.max(probs, axis=-1)
    idx = jnp.argmax(probs, axis=-1).astype(jnp.int32)

    mask = jax.nn.one_hot(idx, E, dtype=jnp.int32)
    pos_all = jnp.cumsum(mask, axis=0) - 1
    pos = jnp.sum(pos_all * mask, axis=1)
    kept = pos < CAP
    tok = jnp.arange(N, dtype=jnp.int32)
    slot = idx * CAP + pos
    slot_scatter = jnp.where(kept, slot, R + tok // TPW)
    slot_gather = jnp.where(kept, slot, 0)

    counts = jnp.sum(mask, axis=0)
    tokens_per_expert = jnp.minimum(counts, CAP).astype(jnp.float32)
    frac_tok = tokens_per_expert / N
    frac_prob = probs.mean(axis=0)
    aux = 0.01 * E * jnp.sum(frac_tok * frac_prob)

    buf = _dispatch(xf, slot_scatter.reshape(N // CHUNK, CHUNK))
    eo = _ffn(buf, W1, b1.reshape(E, 1, DFF), W2, b2.reshape(E, 1, DIM))

    gate_kept = jnp.where(kept, gating, 0.0)
    grep = jnp.broadcast_to(gate_kept[:, None], (N, LANES))
    krep = jnp.broadcast_to(kept.astype(jnp.int32)[:, None], (N, LANES))
    out = _combine(eo, xf, slot_gather.reshape(N // CCH, CCH), grep, krep)
    return out.reshape(x.shape), aux


# speedup vs baseline: 5.1578x; 1.0756x over previous
import functools

import jax
import jax.numpy as jnp
from jax import lax
from jax.experimental import pallas as pl
from jax.experimental.pallas import tpu as pltpu
from jax.experimental.pallas import tpu_sc as plsc

DIM = 2048
E = 8
DFF = 8192
N = 8192
CAP = 1280
R = E * CAP
NW = 32
TPW = N // NW
CHUNK = 8
NCH = TPW // CHUNK
CCH = 4
NCC = TPW // CCH
RPAD = R + NW
FBLK = 512
LANES = 16

_sc_mesh = plsc.VectorSubcoreMesh(core_axis_name="c", subcore_axis_name="s")


def _router_body(x_ref, wr_ref, out_ref):
    lg = lax.dot_general(x_ref[...], wr_ref[...], (((1,), (1,)), ((), ())),
                         preferred_element_type=jnp.float32)
    lane = lax.broadcasted_iota(jnp.int32, lg.shape, 1)
    out_ref[...] = jnp.where(lane < E, lg, -1e30)


def _router(xf, wr_pad):
    tb = 1024
    return pl.pallas_call(
        _router_body,
        grid=(N // tb,),
        in_specs=[pl.BlockSpec((tb, DIM), lambda i: (i, 0)),
                  pl.BlockSpec((128, DIM), lambda i: (0, 0))],
        out_specs=pl.BlockSpec((tb, 128), lambda i: (i, 0)),
        out_shape=jax.ShapeDtypeStruct((N, 128), jnp.float32),
    )(xf, wr_pad)


@functools.partial(
    pl.kernel,
    out_type=jax.ShapeDtypeStruct((RPAD, DIM), jnp.float32),
    mesh=_sc_mesh,
    scratch_types=[pltpu.VMEM((NCH, CHUNK), jnp.int32),
                   pltpu.VMEM((CHUNK, DIM), jnp.float32),
                   pltpu.VMEM((CHUNK, DIM), jnp.float32),
                   pltpu.SemaphoreType.DMA,
                   pltpu.SemaphoreType.DMA,
                   pltpu.SemaphoreType.DMA,
                   pltpu.SemaphoreType.DMA],
)
def _dispatch(xf_hbm, slot_hbm, buf_hbm, idx_v, r0, r1, l0, l1, s0, s1):
    wid = lax.axis_index("s") * 2 + lax.axis_index("c")
    pltpu.sync_copy(slot_hbm.at[pl.ds(wid * NCH, NCH)], idx_v)
    rows, lsem, ssem = (r0, r1), (l0, l1), (s0, s1)
    load, scat = [None, None], [None, None]
    load[0] = pltpu.async_copy(
        xf_hbm.at[pl.ds(wid * TPW, CHUNK)], rows[0], lsem[0])
    for c in range(NCH):
        b = c % 2
        if c + 1 < NCH:
            nb = (c + 1) % 2
            if scat[nb] is not None:
                scat[nb].wait()
            load[nb] = pltpu.async_copy(
                xf_hbm.at[pl.ds(wid * TPW + (c + 1) * CHUNK, CHUNK)],
                rows[nb], lsem[nb])
        load[b].wait()
        scat[b] = pltpu.async_copy(rows[b], buf_hbm.at[idx_v.at[c]], ssem[b])
    scat[0].wait()
    scat[1].wait()


def _ffn_body(x_ref, w1_ref, b1_ref, w2_ref, b2_ref, out_ref):
    f = pl.program_id(1)

    @pl.when(f == 0)
    def _():
        out_ref[...] = jnp.broadcast_to(b2_ref[0], out_ref.shape)

    h = jnp.dot(x_ref[...], w1_ref[0], preferred_element_type=jnp.float32)
    h = jax.nn.gelu(h + b1_ref[0])
    out_ref[...] += jnp.dot(h, w2_ref[0], preferred_element_type=jnp.float32)


def _ffn(buf, w1, b1, w2, b2):
    grid = (E, DFF // FBLK)
    return pl.pallas_call(
        _ffn_body,
        grid=grid,
        in_specs=[
            pl.BlockSpec((CAP, DIM), lambda e, f: (e, 0)),
            pl.BlockSpec((1, DIM, FBLK), lambda e, f: (e, 0, f)),
            pl.BlockSpec((1, 1, FBLK), lambda e, f: (e, 0, f)),
            pl.BlockSpec((1, FBLK, DIM), lambda e, f: (e, f, 0)),
            pl.BlockSpec((1, 1, DIM), lambda e, f: (e, 0, 0)),
        ],
        out_specs=pl.BlockSpec((CAP, DIM), lambda e, f: (e, 0)),
        out_shape=jax.ShapeDtypeStruct((R, DIM), jnp.float32),
        compiler_params=pltpu.CompilerParams(
            dimension_semantics=("parallel", "arbitrary"),
            vmem_limit_bytes=63 * 1024 * 1024),
    )(buf, w1, b1, w2, b2)


@functools.partial(
    pl.kernel,
    out_type=jax.ShapeDtypeStruct((N, DIM), jnp.float32),
    mesh=_sc_mesh,
    scratch_types=[pltpu.VMEM((NCC, CCH), jnp.int32),
                   pltpu.VMEM((TPW, LANES), jnp.float32),
                   pltpu.VMEM((TPW, LANES), jnp.int32),
                   pltpu.VMEM((CCH, DIM), jnp.float32),
                   pltpu.VMEM((CCH, DIM), jnp.float32),
                   pltpu.VMEM((CCH, DIM), jnp.float32),
                   pltpu.VMEM((CCH, DIM), jnp.float32),
                   pltpu.SemaphoreType.DMA,
                   pltpu.SemaphoreType.DMA,
                   pltpu.SemaphoreType.DMA,
                   pltpu.SemaphoreType.DMA,
                   pltpu.SemaphoreType.DMA,
                   pltpu.SemaphoreType.DMA],
)
def _combine(eo_hbm, xf_hbm, gslot_hbm, grep_hbm, krep_hbm, out_hbm,
             idx_v, g_v, k_v, e0, e1, x0, x1, ge0, ge1, lx0, lx1, st0, st1):
    wid = lax.axis_index("s") * 2 + lax.axis_index("c")
    pltpu.sync_copy(gslot_hbm.at[pl.ds(wid * NCC, NCC)], idx_v)
    pltpu.sync_copy(grep_hbm.at[pl.ds(wid * TPW, TPW)], g_v)
    pltpu.sync_copy(krep_hbm.at[pl.ds(wid * TPW, TPW)], k_v)
    eob, xfb = (e0, e1), (x0, x1)
    gsem, xsem, osem = (ge0, ge1), (lx0, lx1), (st0, st1)
    gath, load, stor = [None, None], [None, None], [None, None]
    gath[0] = pltpu.async_copy(eo_hbm.at[idx_v.at[0]], eob[0], gsem[0])
    load[0] = pltpu.async_copy(
        xf_hbm.at[pl.ds(wid * TPW, CCH)], xfb[0], xsem[0])
    for c in range(NCC):
        b = c % 2
        base = wid * TPW + c * CCH
        if c + 1 < NCC:
            nb = (c + 1) % 2
            if stor[nb] is not None:
                stor[nb].wait()
            gath[nb] = pltpu.async_copy(
                eo_hbm.at[idx_v.at[c + 1]], eob[nb], gsem[nb])
            load[nb] = pltpu.async_copy(
                xf_hbm.at[pl.ds(base + CCH, CCH)], xfb[nb], xsem[nb])
        gath[b].wait()
        load[b].wait()
        eo_v, xf_v = eob[b], xfb[b]

        def tok_body(j, carry):
            g16 = g_v[c * CCH + j]
            k16 = k_v[c * CCH + j] > 0

            def col_body(col, inner):
                sl = pl.ds(col * LANES, LANES)
                e16 = eo_v[j, sl]
                x16 = xf_v[j, sl]
                xf_v[j, sl] = jnp.where(k16, g16 * e16, x16)
                return inner

            lax.fori_loop(0, DIM // LANES, col_body, 0, unroll=8)
            return carry

        lax.fori_loop(0, CCH, tok_body, 0)
        stor[b] = pltpu.async_copy(
            xfb[b], out_hbm.at[pl.ds(base, CCH)], osem[b])
    stor[0].wait()
    stor[1].wait()


def kernel(x, Wr, W1, b1, W2, b2):
    xf = x.reshape(N, DIM)

    wr_pad = jnp.zeros((128, DIM), jnp.float32).at[:E].set(Wr)
    logits = _router(xf, wr_pad)
    probs = jax.nn.softmax(logits, axis=-1)[:, :E]
    gating = jnp.max(probs, axis=-1)
    idx = jnp.argmax(probs, axis=-1).astype(jnp.int32)

    mask = jax.nn.one_hot(idx, E, dtype=jnp.int32)
    pos_all = jnp.cumsum(mask, axis=0) - 1
    pos = jnp.sum(pos_all * mask, axis=1)
    kept = pos < CAP
    tok = jnp.arange(N, dtype=jnp.int32)
    slot = idx * CAP + pos
    slot_scatter = jnp.where(kept, slot, R + tok // TPW)
    slot_gather = jnp.where(kept, slot, 0)

    counts = jnp.sum(mask, axis=0)
    tokens_per_expert = jnp.minimum(counts, CAP).astype(jnp.float32)
    frac_tok = tokens_per_expert / N
    frac_prob = probs.mean(axis=0)
    aux = 0.01 * E * jnp.sum(frac_tok * frac_prob)

    buf = _dispatch(xf, slot_scatter.reshape(N // CHUNK, CHUNK))
    eo = _ffn(buf, W1, b1.reshape(E, 1, DFF), W2, b2.reshape(E, 1, DIM))

    gate_kept = jnp.where(kept, gating, 0.0)
    grep = jnp.broadcast_to(gate_kept[:, None], (N, LANES))
    krep = jnp.broadcast_to(kept.astype(jnp.int32)[:, None], (N, LANES))
    out = _combine(eo, xf, slot_gather.reshape(N // CCH, CCH), grep, krep)
    return out.reshape(x.shape), aux
